# baseline (speedup 1.0000x reference)
import functools

import jax
import jax.numpy as jnp
from jax import lax
from jax.experimental import pallas as pl
from jax.experimental.pallas import tpu as pltpu
from jax.experimental.pallas import tpu_sc as plsc

NC = 2
NS = 16
NW = NC * NS
G = 64
NBUF = 2
NIB = 3


def _sc_join(unary, binary, idx1g, idx2g):
    V, D = unary.shape
    B, E = binary.shape
    NG = idx1g.shape[0]
    W = 2 * D + E
    base_pw = NG // NW
    rem = NG - base_pw * NW
    assert base_pw % 6 == 0 and base_pw >= 12

    mesh = plsc.VectorSubcoreMesh(core_axis_name="c", subcore_axis_name="s")

    @functools.partial(
        pl.kernel,
        out_type=jax.ShapeDtypeStruct((B, W), jnp.float32),
        mesh=mesh,
        scratch_types=[
            pltpu.VMEM((NIB, G), jnp.int32),
            pltpu.VMEM((NIB, G), jnp.int32),
            pltpu.VMEM((NBUF, G, D), jnp.float32),
            pltpu.VMEM((NBUF, G, D), jnp.float32),
            pltpu.VMEM((NBUF, G, E), jnp.float32),
            [pltpu.SemaphoreType.DMA] * NIB,
            [pltpu.SemaphoreType.DMA] * NBUF,
            [pltpu.SemaphoreType.DMA] * NBUF,
        ],
    )
    def join_kernel(unary_h, binary_h, idx1_h, idx2_h, out_h,
                    i1_v, i2_v, r1_v, r2_v, b_v,
                    sem_idx, sem_in, sem_out):
        cid = lax.axis_index("c")
        sid = lax.axis_index("s")
        wid = sid * NC + cid
        g0 = wid * base_pw

        def issue_idx(g, k):
            pltpu.async_copy(idx1_h.at[g0 + g], i1_v.at[k], sem_idx[k])
            pltpu.async_copy(idx2_h.at[g0 + g], i2_v.at[k], sem_idx[k])

        def wait_idx(k):
            pltpu.make_async_copy(idx1_h.at[0], i1_v.at[k], sem_idx[k]).wait()
            pltpu.make_async_copy(idx2_h.at[0], i2_v.at[k], sem_idx[k]).wait()

        def issue_in(g, b, k):
            pltpu.async_copy(unary_h.at[i1_v.at[k]], r1_v.at[b], sem_in[b])
            pltpu.async_copy(unary_h.at[i2_v.at[k]], r2_v.at[b], sem_in[b])
            pltpu.async_copy(binary_h.at[pl.ds((g0 + g) * G, G)], b_v.at[b],
                             sem_in[b])

        def wait_in(b):
            pltpu.make_async_copy(unary_h.at[pl.ds(0, G)], r1_v.at[b],
                                  sem_in[b]).wait()
            pltpu.make_async_copy(unary_h.at[pl.ds(0, G)], r2_v.at[b],
                                  sem_in[b]).wait()
            pltpu.make_async_copy(binary_h.at[pl.ds(0, G)], b_v.at[b],
                                  sem_in[b]).wait()

        def issue_out(g, b):
            row = (g0 + g) * G
            pltpu.async_copy(r1_v.at[b], out_h.at[pl.ds(row, G), pl.ds(0, D)],
                             sem_out[b])
            pltpu.async_copy(r2_v.at[b], out_h.at[pl.ds(row, G), pl.ds(D, D)],
                             sem_out[b])
            pltpu.async_copy(b_v.at[b], out_h.at[pl.ds(row, G), pl.ds(2 * D, E)],
                             sem_out[b])

        def wait_out(b):
            pltpu.make_async_copy(r1_v.at[b], out_h.at[pl.ds(0, G), pl.ds(0, D)],
                                  sem_out[b]).wait()
            pltpu.make_async_copy(r2_v.at[b], out_h.at[pl.ds(0, G), pl.ds(D, D)],
                                  sem_out[b]).wait()
            pltpu.make_async_copy(b_v.at[b], out_h.at[pl.ds(0, G),
                                                      pl.ds(2 * D, E)],
                                  sem_out[b]).wait()

        def slot(g, p, do_waitout, do_in, do_idx):
            b = p % NBUF
            wait_in(b)
            issue_out(g, b)
            bm = 1 - b
            if do_waitout:
                wait_out(bm)
            if do_in:
                wait_idx((p + 1) % NIB)
                issue_in(g + 1, bm, (p + 1) % NIB)
            if do_idx:
                issue_idx(g + 2, (p + 2) % NIB)

        issue_idx(0, 0)
        issue_idx(1, 1)
        wait_idx(0)
        issue_in(0, 0, 0)

        slot(0, 0, False, True, True)
        for p in range(1, 6):
            slot(p, p, True, True, True)

        @pl.loop(6, base_pw - 6, step=6)
        def _(jj):
            for p in range(6):
                slot(jj + p, p, True, True, True)

        for p in range(6):
            g = base_pw - 6 + p
            slot(g, p, True, p < 5, p < 4)
        wait_out((base_pw - 1) % NBUF)

        @pl.when(wid < rem)
        def _():
            eg = NW * base_pw + wid
            row = eg * G
            pltpu.sync_copy(idx1_h.at[eg], i1_v.at[0])
            pltpu.sync_copy(idx2_h.at[eg], i2_v.at[0])
            c1 = pltpu.async_copy(unary_h.at[i1_v.at[0]], r1_v.at[0], sem_in[0])
            c2 = pltpu.async_copy(unary_h.at[i2_v.at[0]], r2_v.at[0], sem_in[0])
            pltpu.sync_copy(binary_h.at[pl.ds(row, G)], b_v.at[0])
            c1.wait()
            c2.wait()
            pltpu.sync_copy(r1_v.at[0], out_h.at[pl.ds(row, G), pl.ds(0, D)])
            pltpu.sync_copy(r2_v.at[0], out_h.at[pl.ds(row, G), pl.ds(D, D)])
            pltpu.sync_copy(b_v.at[0], out_h.at[pl.ds(row, G), pl.ds(2 * D, E)])

    return join_kernel(unary, binary, idx1g, idx2g)


def kernel(unary, binary, index1, index2):
    B = index1.shape[0]
    idx1g = index1.reshape(B // G, G)
    idx2g = index2.reshape(B // G, G)
    return _sc_join(unary, binary, idx1g, idx2g)


# speedup vs baseline: 1.0355x; 1.0355x over previous
import functools

import jax
import jax.numpy as jnp
from jax import lax
from jax.experimental import pallas as pl
from jax.experimental.pallas import tpu as pltpu
from jax.experimental.pallas import tpu_sc as plsc

NC = 2
NS = 16
NW = NC * NS
G = 80
NBUF = 4


def _sc_join(unary, binary, idx1g, idx2g):
    V, D = unary.shape
    B, E = binary.shape
    NG = idx1g.shape[0]
    W = 2 * D + E
    n_pw = NG // NW - 1
    assert n_pw % NBUF == 0 and n_pw >= 3 * NBUF
    assert NG == NW * (n_pw + 1)

    mesh = plsc.VectorSubcoreMesh(core_axis_name="c", subcore_axis_name="s")

    @functools.partial(
        pl.kernel,
        out_type=jax.ShapeDtypeStruct((B, W), jnp.float32),
        mesh=mesh,
        scratch_types=[
            pltpu.VMEM((NBUF, G), jnp.int32),
            pltpu.VMEM((NBUF, G), jnp.int32),
            pltpu.VMEM((NBUF, G, D), jnp.float32),
            pltpu.VMEM((NBUF, G, D), jnp.float32),
            pltpu.VMEM((NBUF, G, E), jnp.float32),
            [pltpu.SemaphoreType.DMA] * NBUF,
            [pltpu.SemaphoreType.DMA] * NBUF,
            [pltpu.SemaphoreType.DMA] * NBUF,
        ],
    )
    def join_kernel(unary_h, binary_h, idx1_h, idx2_h, out_h,
                    i1_v, i2_v, r1_v, r2_v, b_v,
                    sem_idx, sem_in, sem_out):
        cid = lax.axis_index("c")
        sid = lax.axis_index("s")
        wid = sid * NC + cid
        g0 = wid * n_pw

        def issue_idx(g, k):
            pltpu.async_copy(idx1_h.at[g0 + g], i1_v.at[k], sem_idx[k])
            pltpu.async_copy(idx2_h.at[g0 + g], i2_v.at[k], sem_idx[k])

        def wait_idx(k):
            pltpu.make_async_copy(idx1_h.at[0], i1_v.at[k], sem_idx[k]).wait()
            pltpu.make_async_copy(idx2_h.at[0], i2_v.at[k], sem_idx[k]).wait()

        def issue_in(g, b):
            pltpu.async_copy(unary_h.at[i1_v.at[b]], r1_v.at[b], sem_in[b])
            pltpu.async_copy(unary_h.at[i2_v.at[b]], r2_v.at[b], sem_in[b])
            pltpu.async_copy(binary_h.at[pl.ds((g0 + g) * G, G)], b_v.at[b],
                             sem_in[b])

        def wait_in(b):
            pltpu.make_async_copy(unary_h.at[pl.ds(0, G)], r1_v.at[b],
                                  sem_in[b]).wait()
            pltpu.make_async_copy(unary_h.at[pl.ds(0, G)], r2_v.at[b],
                                  sem_in[b]).wait()
            pltpu.make_async_copy(binary_h.at[pl.ds(0, G)], b_v.at[b],
                                  sem_in[b]).wait()

        def issue_out(g, b):
            row = (g0 + g) * G
            pltpu.async_copy(r1_v.at[b], out_h.at[pl.ds(row, G), pl.ds(0, D)],
                             sem_out[b])
            pltpu.async_copy(r2_v.at[b], out_h.at[pl.ds(row, G), pl.ds(D, D)],
                             sem_out[b])
            pltpu.async_copy(b_v.at[b], out_h.at[pl.ds(row, G), pl.ds(2 * D, E)],
                             sem_out[b])

        def wait_out(b):
            pltpu.make_async_copy(r1_v.at[b], out_h.at[pl.ds(0, G), pl.ds(0, D)],
                                  sem_out[b]).wait()
            pltpu.make_async_copy(r2_v.at[b], out_h.at[pl.ds(0, G), pl.ds(D, D)],
                                  sem_out[b]).wait()
            pltpu.make_async_copy(b_v.at[b], out_h.at[pl.ds(0, G),
                                                      pl.ds(2 * D, E)],
                                  sem_out[b]).wait()

        def slot(g, p, do_waitout, do_in, do_idx):
            b = p
            b2 = (p + 2) % NBUF
            wait_in(b)
            issue_out(g, b)
            if do_in:
                wait_idx(b2)
                if do_waitout:
                    wait_out(b2)
                issue_in(g + 2, b2)
            if do_idx:
                issue_idx(g + 4, b)

        issue_idx(0, 0)
        issue_idx(1, 1)
        wait_idx(0)
        issue_in(0, 0)
        issue_idx(2, 2)
        wait_idx(1)
        issue_in(1, 1)
        issue_idx(3, 3)

        slot(0, 0, False, True, True)
        slot(1, 1, False, True, True)

        @pl.loop(2, n_pw - 6, step=NBUF)
        def _(jj):
            for p in range(NBUF):
                slot(jj + p, (2 + p) % NBUF, True, True, True)

        for i in range(6):
            g = n_pw - 6 + i
            slot(g, g % NBUF, True, i < 4, i < 2)
        for i in range(NBUF):
            wait_out((n_pw - 4 + i) % NBUF)

        eg = NW * n_pw + wid
        row = eg * G
        pltpu.sync_copy(idx1_h.at[eg], i1_v.at[0])
        pltpu.sync_copy(idx2_h.at[eg], i2_v.at[0])
        c1 = pltpu.async_copy(unary_h.at[i1_v.at[0]], r1_v.at[0], sem_in[0])
        c2 = pltpu.async_copy(unary_h.at[i2_v.at[0]], r2_v.at[0], sem_in[0])
        pltpu.sync_copy(binary_h.at[pl.ds(row, G)], b_v.at[0])
        c1.wait()
        c2.wait()
        pltpu.sync_copy(r1_v.at[0], out_h.at[pl.ds(row, G), pl.ds(0, D)])
        pltpu.sync_copy(r2_v.at[0], out_h.at[pl.ds(row, G), pl.ds(D, D)])
        pltpu.sync_copy(b_v.at[0], out_h.at[pl.ds(row, G), pl.ds(2 * D, E)])

    return join_kernel(unary, binary, idx1g, idx2g)


def kernel(unary, binary, index1, index2):
    B = index1.shape[0]
    idx1g = index1.reshape(B // G, G)
    idx2g = index2.reshape(B // G, G)
    return _sc_join(unary, binary, idx1g, idx2g)


# speedup vs baseline: 1.0444x; 1.0086x over previous
import functools

import jax
import jax.numpy as jnp
from jax import lax
from jax.experimental import pallas as pl
from jax.experimental.pallas import tpu as pltpu
from jax.experimental.pallas import tpu_sc as plsc

NC = 2
NS = 16
NW = NC * NS
G = 80
NBUF = 4


def _sc_join(unary, binary, idx1, idx2):
    V, D = unary.shape
    B, E = binary.shape
    NG = B // G
    W = 2 * D + E
    n_pw = NG // NW - 1
    assert n_pw % NBUF == 0 and n_pw >= 3 * NBUF
    assert NG == NW * (n_pw + 1)

    mesh = plsc.VectorSubcoreMesh(core_axis_name="c", subcore_axis_name="s")

    @functools.partial(
        pl.kernel,
        out_type=jax.ShapeDtypeStruct((B, W), jnp.float32),
        mesh=mesh,
        compiler_params=pltpu.CompilerParams(use_tc_tiling_on_sc=True),
        scratch_types=[
            pltpu.VMEM((NBUF, G), jnp.int32),
            pltpu.VMEM((NBUF, G), jnp.int32),
            pltpu.VMEM((NBUF, G, D), jnp.float32),
            pltpu.VMEM((NBUF, G, D), jnp.float32),
            pltpu.VMEM((NBUF, G, E), jnp.float32),
            [pltpu.SemaphoreType.DMA] * NBUF,
            [pltpu.SemaphoreType.DMA] * NBUF,
            [pltpu.SemaphoreType.DMA] * NBUF,
        ],
    )
    def join_kernel(unary_h, binary_h, idx1_h, idx2_h, out_h,
                    i1_v, i2_v, r1_v, r2_v, b_v,
                    sem_idx, sem_in, sem_out):
        cid = lax.axis_index("c")
        sid = lax.axis_index("s")
        wid = sid * NC + cid
        g0 = wid * n_pw

        def issue_idx(g, k):
            pltpu.async_copy(idx1_h.at[pl.ds((g0 + g) * G, G)], i1_v.at[k],
                             sem_idx[k])
            pltpu.async_copy(idx2_h.at[pl.ds((g0 + g) * G, G)], i2_v.at[k],
                             sem_idx[k])

        def wait_idx(k):
            pltpu.make_async_copy(idx1_h.at[pl.ds(0, G)], i1_v.at[k],
                                  sem_idx[k]).wait()
            pltpu.make_async_copy(idx2_h.at[pl.ds(0, G)], i2_v.at[k],
                                  sem_idx[k]).wait()

        def issue_in(g, b):
            pltpu.async_copy(unary_h.at[i1_v.at[b]], r1_v.at[b], sem_in[b])
            pltpu.async_copy(unary_h.at[i2_v.at[b]], r2_v.at[b], sem_in[b])
            pltpu.async_copy(binary_h.at[pl.ds((g0 + g) * G, G)], b_v.at[b],
                             sem_in[b])

        def wait_in(b):
            pltpu.make_async_copy(unary_h.at[pl.ds(0, G)], r1_v.at[b],
                                  sem_in[b]).wait()
            pltpu.make_async_copy(unary_h.at[pl.ds(0, G)], r2_v.at[b],
                                  sem_in[b]).wait()
            pltpu.make_async_copy(binary_h.at[pl.ds(0, G)], b_v.at[b],
                                  sem_in[b]).wait()

        def issue_out(g, b):
            row = (g0 + g) * G
            pltpu.async_copy(r1_v.at[b], out_h.at[pl.ds(row, G), pl.ds(0, D)],
                             sem_out[b])
            pltpu.async_copy(r2_v.at[b], out_h.at[pl.ds(row, G), pl.ds(D, D)],
                             sem_out[b])
            pltpu.async_copy(b_v.at[b], out_h.at[pl.ds(row, G), pl.ds(2 * D, E)],
                             sem_out[b])

        def wait_out(b):
            pltpu.make_async_copy(r1_v.at[b], out_h.at[pl.ds(0, G), pl.ds(0, D)],
                                  sem_out[b]).wait()
            pltpu.make_async_copy(r2_v.at[b], out_h.at[pl.ds(0, G), pl.ds(D, D)],
                                  sem_out[b]).wait()
            pltpu.make_async_copy(b_v.at[b], out_h.at[pl.ds(0, G),
                                                      pl.ds(2 * D, E)],
                                  sem_out[b]).wait()

        def slot(g, p, do_waitout, do_in, do_idx):
            b = p
            b2 = (p + 2) % NBUF
            wait_in(b)
            issue_out(g, b)
            if do_in:
                wait_idx(b2)
                if do_waitout:
                    wait_out(b2)
                issue_in(g + 2, b2)
            if do_idx:
                issue_idx(g + 4, b)

        issue_idx(0, 0)
        issue_idx(1, 1)
        wait_idx(0)
        issue_in(0, 0)
        issue_idx(2, 2)
        wait_idx(1)
        issue_in(1, 1)
        issue_idx(3, 3)

        slot(0, 0, False, True, True)
        slot(1, 1, False, True, True)

        @pl.loop(2, n_pw - 6, step=NBUF)
        def _(jj):
            for p in range(NBUF):
                slot(jj + p, (2 + p) % NBUF, True, True, True)

        for i in range(6):
            g = n_pw - 6 + i
            slot(g, g % NBUF, True, i < 4, i < 2)
        for i in range(NBUF):
            wait_out((n_pw - 4 + i) % NBUF)

        eg = NW * n_pw + wid
        row = eg * G
        pltpu.sync_copy(idx1_h.at[pl.ds(row, G)], i1_v.at[0])
        pltpu.sync_copy(idx2_h.at[pl.ds(row, G)], i2_v.at[0])
        c1 = pltpu.async_copy(unary_h.at[i1_v.at[0]], r1_v.at[0], sem_in[0])
        c2 = pltpu.async_copy(unary_h.at[i2_v.at[0]], r2_v.at[0], sem_in[0])
        pltpu.sync_copy(binary_h.at[pl.ds(row, G)], b_v.at[0])
        c1.wait()
        c2.wait()
        pltpu.sync_copy(r1_v.at[0], out_h.at[pl.ds(row, G), pl.ds(0, D)])
        pltpu.sync_copy(r2_v.at[0], out_h.at[pl.ds(row, G), pl.ds(D, D)])
        pltpu.sync_copy(b_v.at[0], out_h.at[pl.ds(row, G), pl.ds(2 * D, E)])

    return join_kernel(unary, binary, idx1, idx2)


def kernel(unary, binary, index1, index2):
    return _sc_join(unary, binary, index1, index2)
